# Initial kernel scaffold; baseline (speedup 1.0000x reference)
import jax
import jax.numpy as jnp
from jax.experimental import pallas as pl


def kernel(act_ids, res_ids, num_feats, act_table, res_table, ln1_g, ln1_b, W, b, ln2_g, ln2_b):
    raise NotImplementedError("write your pallas kernel here")


# speedup vs baseline: 2.0353x; 2.0353x over previous
import functools

import jax
import jax.numpy as jnp
from jax import lax
from jax.experimental import pallas as pl
from jax.experimental.pallas import tpu as pltpu
from jax.experimental.pallas import tpu_sc as plsc

_NC = 2
_NS = 16
_NW = _NC * _NS


def _make_gather(n, da, dr):
    bpw = n // _NW
    ch = 1024
    grp = bpw // ch
    k = ch // 128

    mesh = plsc.VectorSubcoreMesh(core_axis_name="c", subcore_axis_name="s")

    @functools.partial(
        pl.kernel,
        mesh=mesh,
        compiler_params=pltpu.CompilerParams(use_tc_tiling_on_sc=False),
        out_type=(
            jax.ShapeDtypeStruct((n, da), jnp.float32),
            jax.ShapeDtypeStruct((n, dr), jnp.float32),
        ),
        scratch_types=[
            pltpu.VMEM((k, 128), jnp.int32),
            pltpu.VMEM((k, 128), jnp.int32),
            pltpu.VMEM((ch, da), jnp.float32),
            pltpu.VMEM((ch, dr), jnp.float32),
            pltpu.SemaphoreType.DMA,
            pltpu.SemaphoreType.DMA,
        ],
    )
    def gather_k(aid_hbm, rid_hbm, at_hbm, rt_hbm, aout, rout,
                 aidx, ridx, arows, rrows, sema, semr):
        wid = lax.axis_index("s") * _NC + lax.axis_index("c")

        def body(g, carry):
            base = pl.multiple_of(wid * bpw + g * ch, ch)
            rb = pl.multiple_of(base // 128, ch // 128)
            pltpu.sync_copy(aid_hbm.at[pl.ds(rb, k)], aidx)
            pltpu.sync_copy(rid_hbm.at[pl.ds(rb, k)], ridx)
            handles = []
            for j in range(k):
                handles.append(pltpu.async_copy(
                    at_hbm.at[aidx.at[j]], arows.at[pl.ds(j * 128, 128)], sema))
                handles.append(pltpu.async_copy(
                    rt_hbm.at[ridx.at[j]], rrows.at[pl.ds(j * 128, 128)], semr))
            for h in handles:
                h.wait()
            pltpu.sync_copy(arows, aout.at[pl.ds(base, ch)])
            pltpu.sync_copy(rrows, rout.at[pl.ds(base, ch)])
            return carry

        lax.fori_loop(0, grp, body, 0)

    return gather_k


def _dense_body(a_ref, r_ref, f_ref, wa_ref, wr_ref, wf_ref, beff_ref,
                g2_ref, b2_ref, o_ref, *, total_in):
    a = a_ref[...]
    r = r_ref[...]
    f = f_ref[...]
    nf = jnp.log1p(jnp.maximum(f, 0.0))
    inv_n = 1.0 / total_in
    s = (jnp.sum(a, 1, keepdims=True) + jnp.sum(r, 1, keepdims=True)
         + jnp.sum(nf, 1, keepdims=True))
    mu = s * inv_n
    da = a - mu
    dr = r - mu
    df = nf - mu
    var = (jnp.sum(da * da, 1, keepdims=True)
           + jnp.sum(dr * dr, 1, keepdims=True)
           + jnp.sum(df * df, 1, keepdims=True)) * inv_n
    inv = lax.rsqrt(var + 1e-5)
    y = jnp.dot(da, wa_ref[...], preferred_element_type=jnp.float32)
    y = y + jnp.dot(dr, wr_ref[...], preferred_element_type=jnp.float32)
    wf = wf_ref[...]
    y = (y + df[:, 0:1] * wf[0:1, :] + df[:, 1:2] * wf[1:2, :]
         + df[:, 2:3] * wf[2:3, :])
    y = y * inv + beff_ref[...]
    yg = 0.5 * y * (1.0 + lax.erf(y * 0.7071067811865476))
    mu2 = jnp.mean(yg, 1, keepdims=True)
    d2 = yg - mu2
    var2 = jnp.mean(d2 * d2, 1, keepdims=True)
    o_ref[...] = d2 * lax.rsqrt(var2 + 1e-5) * g2_ref[...] + b2_ref[...]


def _dense(act_emb, res_emb, nf, wa, wr, wf, beff, g2, b2, total_in):
    n, da = act_emb.shape
    dr = res_emb.shape[1]
    nfd = nf.shape[1]
    dm = wa.shape[1]
    rblk = 1024
    grid = (n // rblk,)
    return pl.pallas_call(
        functools.partial(_dense_body, total_in=total_in),
        grid=grid,
        in_specs=[
            pl.BlockSpec((rblk, da), lambda i: (i, 0)),
            pl.BlockSpec((rblk, dr), lambda i: (i, 0)),
            pl.BlockSpec((rblk, nfd), lambda i: (i, 0)),
            pl.BlockSpec((da, dm), lambda i: (0, 0)),
            pl.BlockSpec((dr, dm), lambda i: (0, 0)),
            pl.BlockSpec((8, dm), lambda i: (0, 0)),
            pl.BlockSpec((1, dm), lambda i: (0, 0)),
            pl.BlockSpec((1, dm), lambda i: (0, 0)),
            pl.BlockSpec((1, dm), lambda i: (0, 0)),
        ],
        out_specs=pl.BlockSpec((rblk, dm), lambda i: (i, 0)),
        out_shape=jax.ShapeDtypeStruct((n, dm), jnp.float32),
    )(act_emb, res_emb, nf, wa, wr, wf, beff, g2, b2)


def kernel(act_ids, res_ids, num_feats, act_table, res_table,
           ln1_g, ln1_b, W, b, ln2_g, ln2_b):
    n = act_ids.shape[0]
    da = act_table.shape[1]
    dr = res_table.shape[1]
    nfd = num_feats.shape[1]
    dm = W.shape[1]

    aid2 = act_ids.astype(jnp.int32).reshape(n // 128, 128)
    rid2 = res_ids.astype(jnp.int32).reshape(n // 128, 128)
    act_emb, res_emb = _make_gather(n, da, dr)(aid2, rid2, act_table, res_table)

    wg = W * ln1_g[:, None]
    wa = wg[0:da]
    wr = wg[da:da + dr]
    wf = jnp.zeros((8, dm), jnp.float32).at[0:nfd].set(wg[da + dr:])
    beff = (ln1_b @ W + b).reshape(1, dm)
    g2 = ln2_g.reshape(1, dm)
    b2 = ln2_b.reshape(1, dm)
    return _dense(act_emb, res_emb, num_feats, wa, wr, wf, beff, g2, b2,
                  float(da + dr + nfd))


# speedup vs baseline: 2.9474x; 1.4482x over previous
import functools

import jax
import jax.numpy as jnp
from jax import lax
from jax.experimental import pallas as pl
from jax.experimental.pallas import tpu as pltpu
from jax.experimental.pallas import tpu_sc as plsc

_NC = 2
_NS = 16
_NW = _NC * _NS


def _make_gather(n, da, dr):
    half = n // 2
    bpw = half // _NW
    ch = 512
    grp = bpw // ch
    k = ch // 128

    mesh = plsc.VectorSubcoreMesh(core_axis_name="c", subcore_axis_name="s")

    @functools.partial(
        pl.kernel,
        mesh=mesh,
        compiler_params=pltpu.CompilerParams(use_tc_tiling_on_sc=False),
        out_type=jax.ShapeDtypeStruct((half, 128), jnp.float32),
        scratch_types=[
            pltpu.VMEM((k, 128), jnp.int32),
            pltpu.VMEM((k, 128), jnp.int32),
            pltpu.VMEM((k, 128), jnp.int32),
            pltpu.VMEM((k, 128), jnp.int32),
            pltpu.VMEM((ch, da), jnp.float32),
            pltpu.VMEM((ch, da), jnp.float32),
            pltpu.VMEM((ch, dr), jnp.float32),
            pltpu.VMEM((ch, dr), jnp.float32),
            pltpu.SemaphoreType.DMA,
            pltpu.SemaphoreType.DMA,
        ],
    )
    def gather_k(aid_hbm, rid_hbm, at_hbm, rt_hbm, out,
                 aidx1, aidx2, ridx1, ridx2, a1, a2, r1, r2, sema, semr):
        wid = lax.axis_index("s") * _NC + lax.axis_index("c")

        def body(g, carry):
            base = pl.multiple_of(wid * bpw + g * ch, ch)
            rb1 = pl.multiple_of(base // 128, k)
            rb2 = rb1 + half // 128
            pltpu.sync_copy(aid_hbm.at[pl.ds(rb1, k)], aidx1)
            pltpu.sync_copy(aid_hbm.at[pl.ds(rb2, k)], aidx2)
            pltpu.sync_copy(rid_hbm.at[pl.ds(rb1, k)], ridx1)
            pltpu.sync_copy(rid_hbm.at[pl.ds(rb2, k)], ridx2)
            handles = []
            for j in range(k):
                sl = pl.ds(j * 128, 128)
                handles.append(pltpu.async_copy(
                    at_hbm.at[aidx1.at[j]], a1.at[sl], sema))
                handles.append(pltpu.async_copy(
                    at_hbm.at[aidx2.at[j]], a2.at[sl], sema))
                handles.append(pltpu.async_copy(
                    rt_hbm.at[ridx1.at[j]], r1.at[sl], semr))
                handles.append(pltpu.async_copy(
                    rt_hbm.at[ridx2.at[j]], r2.at[sl], semr))
            for h in handles:
                h.wait()
            rows = pl.ds(base, ch)
            pltpu.sync_copy(a1, out.at[rows, pl.ds(0, da)])
            pltpu.sync_copy(r1, out.at[rows, pl.ds(da, dr)])
            pltpu.sync_copy(a2, out.at[rows, pl.ds(64, da)])
            pltpu.sync_copy(r2, out.at[rows, pl.ds(64 + da, dr)])
            return carry

        lax.fori_loop(0, grp, body, 0)

    return gather_k


def _dense_body(x_ref, f1_ref, f2_ref, wc_ref, wf_ref, beff_ref,
                g2_ref, b2_ref, o_ref, *, total_in, da, dr, dm):
    x = x_ref[...]
    rblk = x.shape[0]
    nfeat = total_in - da - dr
    inv_n = 1.0 / total_in
    lane = lax.broadcasted_iota(jnp.int32, (1, 128), 1)
    m1 = lane < (da + dr)
    m2 = jnp.logical_and(lane >= 64, lane < (64 + da + dr))

    nf1 = jnp.log1p(jnp.maximum(f1_ref[...], 0.0))
    nf2 = jnp.log1p(jnp.maximum(f2_ref[...], 0.0))

    xm1 = jnp.where(m1, x, 0.0)
    xm2 = jnp.where(m2, x, 0.0)
    mu1 = (jnp.sum(xm1, 1, keepdims=True)
           + jnp.sum(nf1, 1, keepdims=True)) * inv_n
    mu2 = (jnp.sum(xm2, 1, keepdims=True)
           + jnp.sum(nf2, 1, keepdims=True)) * inv_n
    d1 = jnp.where(m1, x - mu1, 0.0)
    d2 = jnp.where(m2, x - mu2, 0.0)
    dn1 = nf1 - mu1
    dn2 = nf2 - mu2
    var1 = (jnp.sum(d1 * d1, 1, keepdims=True)
            + jnp.sum(dn1 * dn1, 1, keepdims=True)) * inv_n
    var2 = (jnp.sum(d2 * d2, 1, keepdims=True)
            + jnp.sum(dn2 * dn2, 1, keepdims=True)) * inv_n
    inv1 = lax.rsqrt(var1 + 1e-5)
    inv2 = lax.rsqrt(var2 + 1e-5)

    y = jnp.dot(d1 + d2, wc_ref[...], preferred_element_type=jnp.float32)
    wf = wf_ref[...]
    bc1 = (dn1[:, 0:1] * wf[0:1, :] + dn1[:, 1:2] * wf[1:2, :]
           + dn1[:, 2:3] * wf[2:3, :])
    bc2 = (dn2[:, 0:1] * wf[0:1, :] + dn2[:, 1:2] * wf[1:2, :]
           + dn2[:, 2:3] * wf[2:3, :])
    beff = beff_ref[...]
    g2 = g2_ref[...]
    b2 = b2_ref[...]
    inv_dm = 1.0 / dm
    for h, (ysl, bc, inv) in enumerate((
            (y[:, 0:dm], bc1, inv1), (y[:, dm:2 * dm], bc2, inv2))):
        yh = (ysl + bc) * inv + beff
        yg = 0.5 * yh * (1.0 + lax.erf(yh * 0.7071067811865476))
        mu = jnp.sum(yg, 1, keepdims=True) * inv_dm
        d = yg - mu
        var = jnp.sum(d * d, 1, keepdims=True) * inv_dm
        o_ref[h] = d * lax.rsqrt(var + 1e-5) * g2 + b2
    del nfeat, rblk


def _dense(packed, nf, wc, wf, beff, g2, b2, total_in, da, dr):
    half = packed.shape[0]
    nfd = nf.shape[1]
    dm = beff.shape[1]
    rblk = 1024
    nblk = half // rblk
    out = pl.pallas_call(
        functools.partial(_dense_body, total_in=total_in, da=da, dr=dr,
                          dm=dm),
        grid=(nblk,),
        in_specs=[
            pl.BlockSpec((rblk, 128), lambda i: (i, 0)),
            pl.BlockSpec((rblk, nfd), lambda i: (i, 0)),
            pl.BlockSpec((rblk, nfd), lambda i, _n=nblk: (i + _n, 0)),
            pl.BlockSpec((128, 2 * dm), lambda i: (0, 0)),
            pl.BlockSpec((8, dm), lambda i: (0, 0)),
            pl.BlockSpec((1, dm), lambda i: (0, 0)),
            pl.BlockSpec((1, dm), lambda i: (0, 0)),
            pl.BlockSpec((1, dm), lambda i: (0, 0)),
        ],
        out_specs=pl.BlockSpec((2, rblk, dm), lambda i: (0, i, 0)),
        out_shape=jax.ShapeDtypeStruct((2, half, dm), jnp.float32),
    )(packed, nf, nf, wc, wf, beff, g2, b2)
    return out.reshape(2 * half, dm)


def kernel(act_ids, res_ids, num_feats, act_table, res_table,
           ln1_g, ln1_b, W, b, ln2_g, ln2_b):
    n = act_ids.shape[0]
    da = act_table.shape[1]
    dr = res_table.shape[1]
    nfd = num_feats.shape[1]
    dm = W.shape[1]

    aid2 = act_ids.astype(jnp.int32).reshape(n // 128, 128)
    rid2 = res_ids.astype(jnp.int32).reshape(n // 128, 128)
    packed = _make_gather(n, da, dr)(aid2, rid2, act_table, res_table)

    wg = W * ln1_g[:, None]
    wc = jnp.zeros((128, 2 * dm), jnp.float32)
    wc = wc.at[0:da, 0:dm].set(wg[0:da])
    wc = wc.at[da:da + dr, 0:dm].set(wg[da:da + dr])
    wc = wc.at[64:64 + da, dm:2 * dm].set(wg[0:da])
    wc = wc.at[64 + da:64 + da + dr, dm:2 * dm].set(wg[da:da + dr])
    wf = jnp.zeros((8, dm), jnp.float32).at[0:nfd].set(wg[da + dr:])
    beff = (ln1_b @ W + b).reshape(1, dm)
    g2 = ln2_g.reshape(1, dm)
    b2 = ln2_b.reshape(1, dm)
    return _dense(packed, num_feats, wc, wf, beff, g2, b2,
                  float(da + dr + nfd), da, dr)
